# baseline (speedup 1.0000x reference)
import functools

import jax
import jax.numpy as jnp
from jax import lax
from jax.experimental import pallas as pl
from jax.experimental.pallas import tpu as pltpu
from jax.experimental.pallas import tpu_sc as plsc

MAX_DEG = 64
VOCAB = MAX_DEG + 1
D = 128
N_ROWS = 100000
NUM_CORES = 2
NUM_SUBCORES = 16
NW = NUM_CORES * NUM_SUBCORES
CH = 128
NFULL = N_ROWS // CH
TAIL = N_ROWS - NFULL * CH
CPW = (NFULL + 1 + NW - 1) // NW
PAD_N = CPW * NW * CH
NBUF = 4


def _prep_body(win_ref, wout_ref, ind_ref, outd_ref, wsum_ref, idx_ref):
    win = win_ref[...]
    wout = wout_ref[...]
    wsum_ref[...] = win[:, None, :] + wout[None, :, :]
    a = jnp.clip(ind_ref[...], 0, MAX_DEG)
    b = jnp.clip(outd_ref[...], 0, MAX_DEG)
    idx_ref[...] = a * VOCAB + b


_mesh = plsc.VectorSubcoreMesh(core_axis_name="c", subcore_axis_name="s")


@functools.partial(
    pl.kernel,
    mesh=_mesh,
    out_type=jax.ShapeDtypeStruct((N_ROWS, D), jnp.float32),
    scratch_types=[
        pltpu.VMEM((CH,), jnp.int32),
        pltpu.VMEM((CH,), jnp.int32),
        pltpu.VMEM((CH, D), jnp.float32),
        pltpu.SemaphoreType.DMA,
        pltpu.SemaphoreType.DMA,
    ],
)
def _sc_gather(wsum_hbm, idx_hbm, out_hbm, idx_a, idx_b, rows_v, sem_i, sem_g):
    wid = lax.axis_index("s") * NUM_CORES + lax.axis_index("c")
    slots = (idx_a, idx_b)

    def gbase(c):
        return (c * NW + wid) * CH

    for c in range(CPW):
        g = c * NW + wid
        pltpu.sync_copy(idx_hbm.at[pl.ds(gbase(c), CH)], idx_a)
        pltpu.async_copy(wsum_hbm.at[idx_a], rows_v, sem_g).wait()
        if c < CPW - 1:
            pltpu.sync_copy(rows_v, out_hbm.at[pl.ds(g * CH, CH)])
        else:
            @pl.when(g < NFULL)
            def _():
                pltpu.sync_copy(rows_v, out_hbm.at[pl.ds(g * CH, CH)])

            @pl.when(g == NFULL)
            def _():
                pltpu.sync_copy(rows_v.at[pl.ds(0, TAIL)],
                                out_hbm.at[pl.ds(g * CH, TAIL)])


def kernel(in_degree, out_degree, W_in, W_out):
    pad = PAD_N - N_ROWS
    rows = PAD_N // D
    ind = jnp.pad(in_degree.astype(jnp.int32), (0, pad)).reshape(rows, D)
    outd = jnp.pad(out_degree.astype(jnp.int32), (0, pad)).reshape(rows, D)
    wsum, idxc = pl.pallas_call(
        _prep_body,
        out_shape=[
            jax.ShapeDtypeStruct((VOCAB, VOCAB, D), jnp.float32),
            jax.ShapeDtypeStruct((rows, D), jnp.int32),
        ],
    )(W_in, W_out, ind, outd)
    return _sc_gather(wsum.reshape(VOCAB * VOCAB, D), idxc.reshape(PAD_N))


# speedup vs baseline: 1.8587x; 1.8587x over previous
import functools

import jax
import jax.numpy as jnp
from jax import lax
from jax.experimental import pallas as pl
from jax.experimental.pallas import tpu as pltpu
from jax.experimental.pallas import tpu_sc as plsc

MAX_DEG = 64
VOCAB = MAX_DEG + 1
D = 128
N_ROWS = 100000
NUM_CORES = 2
NUM_SUBCORES = 16
NW = NUM_CORES * NUM_SUBCORES
CH = 128
NFULL = N_ROWS // CH
TAIL = N_ROWS - NFULL * CH
NCH = NFULL + 1
PAD_N = NCH * CH
CPW = (NCH + NW - 1) // NW


def _prep_body(win_ref, wout_ref, ind_ref, outd_ref, wsum_ref, idx_ref):
    win = win_ref[...]
    wout = wout_ref[...]
    wsum_ref[...] = win[:, None, :] + wout[None, :, :]
    a = jnp.clip(ind_ref[...], 0, MAX_DEG)
    b = jnp.clip(outd_ref[...], 0, MAX_DEG)
    idx_ref[...] = a * VOCAB + b


_mesh = plsc.VectorSubcoreMesh(core_axis_name="c", subcore_axis_name="s")


@functools.partial(
    pl.kernel,
    mesh=_mesh,
    out_type=jax.ShapeDtypeStruct((N_ROWS, D), jnp.float32),
    scratch_types=[
        pltpu.VMEM((CH,), jnp.int32),
        pltpu.VMEM((CH, D), jnp.float32),
        pltpu.SemaphoreType.DMA,
    ],
)
def _sc_gather(wsum_hbm, idx_hbm, out_hbm, idx_v, rows_v, sem):
    wid = lax.axis_index("s") * NUM_CORES + lax.axis_index("c")
    for c in range(CPW):
        g = c * NW + wid

        @pl.when(g < NCH)
        def _():
            pltpu.sync_copy(idx_hbm.at[pl.ds(g * CH, CH)], idx_v)
            pltpu.async_copy(wsum_hbm.at[idx_v], rows_v, sem).wait()

        @pl.when(g < NFULL)
        def _():
            pltpu.sync_copy(rows_v, out_hbm.at[pl.ds(g * CH, CH)])

        @pl.when(g == NFULL)
        def _():
            pltpu.sync_copy(rows_v.at[pl.ds(0, TAIL)],
                            out_hbm.at[pl.ds(g * CH, TAIL)])


def kernel(in_degree, out_degree, W_in, W_out):
    pad = PAD_N - N_ROWS
    ind = jnp.pad(in_degree.astype(jnp.int32), (0, pad)).reshape(NCH, CH)
    outd = jnp.pad(out_degree.astype(jnp.int32), (0, pad)).reshape(NCH, CH)
    wsum, idxc = pl.pallas_call(
        _prep_body,
        out_shape=[
            jax.ShapeDtypeStruct((VOCAB, VOCAB, D), jnp.float32),
            jax.ShapeDtypeStruct((NCH, CH), jnp.int32),
        ],
    )(W_in, W_out, ind, outd)
    return _sc_gather(wsum.reshape(VOCAB * VOCAB, D), idxc.reshape(PAD_N))


# speedup vs baseline: 2.1188x; 1.1399x over previous
import functools

import jax
import jax.numpy as jnp
from jax import lax
from jax.experimental import pallas as pl
from jax.experimental.pallas import tpu as pltpu
from jax.experimental.pallas import tpu_sc as plsc

MAX_DEG = 64
VOCAB = MAX_DEG + 1
D = 128
N_ROWS = 100000
NUM_CORES = 2
NUM_SUBCORES = 16
NW = NUM_CORES * NUM_SUBCORES
CH = 128
NFULL = N_ROWS // CH
TAIL = N_ROWS - NFULL * CH
NCH = NFULL + 1
PAD_N = NCH * CH
CPW = (NCH + NW - 1) // NW


def _prep_body(win_ref, wout_ref, ind_ref, outd_ref, wsum_ref, idx_ref):
    win = win_ref[...]
    wout = wout_ref[...]
    wsum_ref[...] = win[:, None, :] + wout[None, :, :]
    a = jnp.clip(ind_ref[...], 0, MAX_DEG)
    b = jnp.clip(outd_ref[...], 0, MAX_DEG)
    idx_ref[...] = a * VOCAB + b


_mesh = plsc.VectorSubcoreMesh(core_axis_name="c", subcore_axis_name="s")


@functools.partial(
    pl.kernel,
    mesh=_mesh,
    out_type=jax.ShapeDtypeStruct((N_ROWS, D), jnp.float32),
    scratch_types=[
        pltpu.VMEM((CH,), jnp.int32),
        pltpu.VMEM((CH, D), jnp.float32),
        pltpu.VMEM((CH, D), jnp.float32),
        pltpu.SemaphoreType.DMA,
        pltpu.SemaphoreType.DMA,
        pltpu.SemaphoreType.DMA,
    ],
)
def _sc_gather(wsum_hbm, idx_hbm, out_hbm, idx_v, rows_a, rows_b,
               sem_g, sem_w0, sem_w1):
    wid = lax.axis_index("s") * NUM_CORES + lax.axis_index("c")
    rows = (rows_a, rows_b)
    sem_w = (sem_w0, sem_w1)
    for c in range(CPW):
        g = c * NW + wid
        b = c & 1

        @pl.when(g < NCH)
        def _(g=g, b=b, c=c):
            pltpu.sync_copy(idx_hbm.at[pl.ds(g * CH, CH)], idx_v)
            if c >= 2:
                pltpu.make_async_copy(
                    rows[b], out_hbm.at[pl.ds((g - 2 * NW) * CH, CH)],
                    sem_w[b]).wait()
            pltpu.async_copy(wsum_hbm.at[idx_v], rows[b], sem_g).wait()

        @pl.when(g < NFULL)
        def _(g=g, b=b):
            pltpu.async_copy(rows[b], out_hbm.at[pl.ds(g * CH, CH)],
                             sem_w[b])

        @pl.when(g == NFULL)
        def _(g=g, b=b):
            pltpu.sync_copy(rows[b].at[pl.ds(0, TAIL)],
                            out_hbm.at[pl.ds(g * CH, TAIL)])

    g23 = (CPW - 2) * NW + wid
    pltpu.make_async_copy(rows[(CPW - 2) & 1],
                          out_hbm.at[pl.ds(g23 * CH, CH)],
                          sem_w[(CPW - 2) & 1]).wait()
    g24 = (CPW - 1) * NW + wid

    @pl.when(g24 < NFULL)
    def _():
        pltpu.make_async_copy(rows[(CPW - 1) & 1],
                              out_hbm.at[pl.ds(g24 * CH, CH)],
                              sem_w[(CPW - 1) & 1]).wait()

    @pl.when(g24 >= NCH)
    def _():
        pltpu.make_async_copy(rows[(CPW - 1) & 1],
                              out_hbm.at[pl.ds((g24 - 2 * NW) * CH, CH)],
                              sem_w[(CPW - 1) & 1]).wait()


def kernel(in_degree, out_degree, W_in, W_out):
    pad = PAD_N - N_ROWS
    ind = jnp.pad(in_degree.astype(jnp.int32), (0, pad)).reshape(NCH, CH)
    outd = jnp.pad(out_degree.astype(jnp.int32), (0, pad)).reshape(NCH, CH)
    wsum, idxc = pl.pallas_call(
        _prep_body,
        out_shape=[
            jax.ShapeDtypeStruct((VOCAB, VOCAB, D), jnp.float32),
            jax.ShapeDtypeStruct((NCH, CH), jnp.int32),
        ],
    )(W_in, W_out, ind, outd)
    return _sc_gather(wsum.reshape(VOCAB * VOCAB, D), idxc.reshape(PAD_N))


# speedup vs baseline: 2.2799x; 1.0760x over previous
import functools

import jax
import jax.numpy as jnp
from jax import lax
from jax.experimental import pallas as pl
from jax.experimental.pallas import tpu as pltpu
from jax.experimental.pallas import tpu_sc as plsc

MAX_DEG = 64
VOCAB = MAX_DEG + 1
D = 128
N_ROWS = 100000
NUM_CORES = 2
NUM_SUBCORES = 16
NW = NUM_CORES * NUM_SUBCORES
CH = 128
NFULL = N_ROWS // CH
TAIL = N_ROWS - NFULL * CH
NCH = NFULL + 1
PAD_N = NCH * CH
CPW = (NCH + NW - 1) // NW


def _prep_body(win_ref, wout_ref, ind_ref, outd_ref, wsum_ref, idx_ref):
    win = win_ref[...]
    wout = wout_ref[...]
    wsum_ref[...] = win[:, None, :] + wout[None, :, :]
    a = jnp.clip(ind_ref[...], 0, MAX_DEG)
    b = jnp.clip(outd_ref[...], 0, MAX_DEG)
    idx_ref[...] = a * VOCAB + b


_mesh = plsc.VectorSubcoreMesh(core_axis_name="c", subcore_axis_name="s")


@functools.partial(
    pl.kernel,
    mesh=_mesh,
    out_type=jax.ShapeDtypeStruct((N_ROWS, D), jnp.float32),
    scratch_types=[
        pltpu.VMEM((CH,), jnp.int32),
        pltpu.VMEM((CH,), jnp.int32),
        pltpu.VMEM((CH, D), jnp.float32),
        pltpu.VMEM((CH, D), jnp.float32),
        pltpu.SemaphoreType.DMA,
        pltpu.SemaphoreType.DMA,
        pltpu.SemaphoreType.DMA,
        pltpu.SemaphoreType.DMA,
    ],
)
def _sc_gather(wsum_hbm, idx_hbm, out_hbm, idx_a, idx_b, rows_a, rows_b,
               sem_i, sem_g, sem_w0, sem_w1):
    wid = lax.axis_index("s") * NUM_CORES + lax.axis_index("c")
    idx = (idx_a, idx_b)
    rows = (rows_a, rows_b)
    sem_w = (sem_w0, sem_w1)
    for c in range(CPW):
        g = c * NW + wid
        b = c & 1

        @pl.when(g < NCH)
        def _(g=g, b=b, c=c):
            if c == 0:
                pltpu.sync_copy(idx_hbm.at[pl.ds(g * CH, CH)], idx[b])
            else:
                pltpu.make_async_copy(
                    idx_hbm.at[pl.ds(g * CH, CH)], idx[b], sem_i).wait()
            if c + 1 < CPW:
                @pl.when(g + NW < NCH)
                def _():
                    pltpu.async_copy(idx_hbm.at[pl.ds((g + NW) * CH, CH)],
                                     idx[1 - b], sem_i)
            if c >= 2:
                pltpu.make_async_copy(
                    rows[b], out_hbm.at[pl.ds((g - 2 * NW) * CH, CH)],
                    sem_w[b]).wait()
            pltpu.async_copy(wsum_hbm.at[idx[b]], rows[b], sem_g).wait()

        @pl.when(g < NFULL)
        def _(g=g, b=b):
            pltpu.async_copy(rows[b], out_hbm.at[pl.ds(g * CH, CH)],
                             sem_w[b])

        @pl.when(g == NFULL)
        def _(g=g, b=b):
            pltpu.sync_copy(rows[b].at[pl.ds(0, TAIL)],
                            out_hbm.at[pl.ds(g * CH, TAIL)])

    g23 = (CPW - 2) * NW + wid
    pltpu.make_async_copy(rows[(CPW - 2) & 1],
                          out_hbm.at[pl.ds(g23 * CH, CH)],
                          sem_w[(CPW - 2) & 1]).wait()
    g24 = (CPW - 1) * NW + wid

    @pl.when(g24 < NFULL)
    def _():
        pltpu.make_async_copy(rows[(CPW - 1) & 1],
                              out_hbm.at[pl.ds(g24 * CH, CH)],
                              sem_w[(CPW - 1) & 1]).wait()

    @pl.when(g24 >= NCH)
    def _():
        pltpu.make_async_copy(rows[(CPW - 1) & 1],
                              out_hbm.at[pl.ds((g24 - 2 * NW) * CH, CH)],
                              sem_w[(CPW - 1) & 1]).wait()


def kernel(in_degree, out_degree, W_in, W_out):
    pad = PAD_N - N_ROWS
    ind = jnp.pad(in_degree.astype(jnp.int32), (0, pad)).reshape(NCH, CH)
    outd = jnp.pad(out_degree.astype(jnp.int32), (0, pad)).reshape(NCH, CH)
    wsum, idxc = pl.pallas_call(
        _prep_body,
        out_shape=[
            jax.ShapeDtypeStruct((VOCAB, VOCAB, D), jnp.float32),
            jax.ShapeDtypeStruct((NCH, CH), jnp.int32),
        ],
    )(W_in, W_out, ind, outd)
    return _sc_gather(wsum.reshape(VOCAB * VOCAB, D), idxc.reshape(PAD_N))


# speedup vs baseline: 2.5256x; 1.1078x over previous
import functools

import jax
import jax.numpy as jnp
from jax import lax
from jax.experimental import pallas as pl
from jax.experimental.pallas import tpu as pltpu
from jax.experimental.pallas import tpu_sc as plsc

MAX_DEG = 64
VOCAB = MAX_DEG + 1
D = 128
N_ROWS = 100000
NUM_CORES = 2
NUM_SUBCORES = 16
NW = NUM_CORES * NUM_SUBCORES
CH = 128
NFULL = N_ROWS // CH
TAIL = N_ROWS - NFULL * CH
NCH = NFULL + 1
PAD_N = NCH * CH
CPW = (NCH + NW - 1) // NW


def _prep_body(win_ref, wout_ref, ind_ref, outd_ref, wsum_ref, idx_ref):
    win = win_ref[...]
    wout = wout_ref[...]
    wsum_ref[...] = win[:, None, :] + wout[None, :, :]
    a = jnp.clip(ind_ref[...], 0, MAX_DEG)
    b = jnp.clip(outd_ref[...], 0, MAX_DEG)
    idx_ref[...] = a * VOCAB + b


_mesh = plsc.VectorSubcoreMesh(core_axis_name="c", subcore_axis_name="s")


@functools.partial(
    pl.kernel,
    mesh=_mesh,
    out_type=jax.ShapeDtypeStruct((N_ROWS, D), jnp.float32),
    scratch_types=[
        pltpu.VMEM((CH,), jnp.int32),
        pltpu.VMEM((CH,), jnp.int32),
        pltpu.VMEM((CH,), jnp.int32),
        pltpu.VMEM((CH, D), jnp.float32),
        pltpu.VMEM((CH, D), jnp.float32),
        pltpu.SemaphoreType.DMA,
        pltpu.SemaphoreType.DMA,
        pltpu.SemaphoreType.DMA,
        pltpu.SemaphoreType.DMA,
        pltpu.SemaphoreType.DMA,
    ],
)
def _sc_gather(wsum_hbm, idx_hbm, out_hbm, idx_a, idx_b, idx_c, rows_a,
               rows_b, sem_i, sem_g0, sem_g1, sem_w0, sem_w1):
    wid = lax.axis_index("s") * NUM_CORES + lax.axis_index("c")
    idx = (idx_a, idx_b, idx_c)
    rows = (rows_a, rows_b)
    sem_g = (sem_g0, sem_g1)
    sem_w = (sem_w0, sem_w1)

    def wait_gather(c):
        pltpu.make_async_copy(wsum_hbm.at[pl.ds(0, CH)], rows[c & 1],
                              sem_g[c & 1]).wait()

    def wait_write(c, g):
        pltpu.make_async_copy(rows[c & 1], out_hbm.at[pl.ds(g * CH, CH)],
                              sem_w[c & 1]).wait()

    pltpu.sync_copy(idx_hbm.at[pl.ds(wid * CH, CH)], idx[0])
    pltpu.async_copy(wsum_hbm.at[idx[0]], rows[0], sem_g[0])
    pltpu.async_copy(idx_hbm.at[pl.ds((NW + wid) * CH, CH)], idx[1], sem_i)

    for c in range(CPW):
        g = c * NW + wid
        b = c & 1
        gn = g + NW

        if c + 1 < CPW:
            @pl.when(gn < NCH)
            def _(c=c, gn=gn, b=b):
                pltpu.make_async_copy(idx_hbm.at[pl.ds(gn * CH, CH)],
                                      idx[(c + 1) % 3], sem_i).wait()
                if c >= 1:
                    wait_write(c - 1, gn - 2 * NW)
                pltpu.async_copy(wsum_hbm.at[idx[(c + 1) % 3]], rows[1 - b],
                                 sem_g[1 - b])
                if c + 2 < CPW:
                    @pl.when(gn + NW < NCH)
                    def _():
                        pltpu.async_copy(
                            idx_hbm.at[pl.ds((gn + NW) * CH, CH)],
                            idx[(c + 2) % 3], sem_i)

        if c < CPW - 1:
            wait_gather(c)
            pltpu.async_copy(rows[b], out_hbm.at[pl.ds(g * CH, CH)],
                             sem_w[b])
        else:
            @pl.when(g < NCH)
            def _(c=c):
                wait_gather(c)

            @pl.when(g < NFULL)
            def _(g=g, b=b):
                pltpu.async_copy(rows[b], out_hbm.at[pl.ds(g * CH, CH)],
                                 sem_w[b])

            @pl.when(g == NFULL)
            def _(g=g, b=b):
                pltpu.sync_copy(rows[b].at[pl.ds(0, TAIL)],
                                out_hbm.at[pl.ds(g * CH, TAIL)])

    g23 = (CPW - 2) * NW + wid
    wait_write(CPW - 2, g23)
    g24 = (CPW - 1) * NW + wid

    @pl.when(g24 < NFULL)
    def _():
        wait_write(CPW - 1, g24)

    @pl.when(g24 >= NCH)
    def _():
        wait_write(CPW - 3, g24 - 2 * NW)


def kernel(in_degree, out_degree, W_in, W_out):
    pad = PAD_N - N_ROWS
    ind = jnp.pad(in_degree.astype(jnp.int32), (0, pad)).reshape(NCH, CH)
    outd = jnp.pad(out_degree.astype(jnp.int32), (0, pad)).reshape(NCH, CH)
    wsum, idxc = pl.pallas_call(
        _prep_body,
        out_shape=[
            jax.ShapeDtypeStruct((VOCAB, VOCAB, D), jnp.float32),
            jax.ShapeDtypeStruct((NCH, CH), jnp.int32),
        ],
    )(W_in, W_out, ind, outd)
    return _sc_gather(wsum.reshape(VOCAB * VOCAB, D), idxc.reshape(PAD_N))


# speedup vs baseline: 2.5529x; 1.0108x over previous
import functools

import jax
import jax.numpy as jnp
from jax import lax
from jax.experimental import pallas as pl
from jax.experimental.pallas import tpu as pltpu
from jax.experimental.pallas import tpu_sc as plsc

MAX_DEG = 64
VOCAB = MAX_DEG + 1
D = 128
N_ROWS = 100000
NUM_CORES = 2
NUM_SUBCORES = 16
NW = NUM_CORES * NUM_SUBCORES
CH = 128
NFULL = N_ROWS // CH
TAIL = N_ROWS - NFULL * CH
NCH = NFULL + 1
PAD_N = NCH * CH
CPW = (NCH + NW - 1) // NW


def _prep_body(win_ref, wout_ref, ind_ref, outd_ref, wsum_ref, idx_ref):
    win = win_ref[...]
    wout = wout_ref[...]
    wsum_ref[...] = win[:, None, :] + wout[None, :, :]
    a = jnp.clip(ind_ref[...], 0, MAX_DEG)
    b = jnp.clip(outd_ref[...], 0, MAX_DEG)
    idx_ref[...] = a * VOCAB + b


_mesh = plsc.VectorSubcoreMesh(core_axis_name="c", subcore_axis_name="s")


@functools.partial(
    pl.kernel,
    mesh=_mesh,
    out_type=jax.ShapeDtypeStruct((N_ROWS, D), jnp.float32),
    scratch_types=[
        pltpu.VMEM((CH,), jnp.int32),
        pltpu.VMEM((CH,), jnp.int32),
        pltpu.VMEM((CH,), jnp.int32),
        pltpu.VMEM((CH, D), jnp.float32),
        pltpu.VMEM((CH, D), jnp.float32),
        pltpu.VMEM((CH, D), jnp.float32),
        pltpu.SemaphoreType.DMA,
        pltpu.SemaphoreType.DMA,
        pltpu.SemaphoreType.DMA,
        pltpu.SemaphoreType.DMA,
        pltpu.SemaphoreType.DMA,
        pltpu.SemaphoreType.DMA,
        pltpu.SemaphoreType.DMA,
    ],
)
def _sc_gather(wsum_hbm, idx_hbm, out_hbm, idx_a, idx_b, idx_c, rows_a,
               rows_b, rows_c, sem_i, sem_g0, sem_g1, sem_g2, sem_w0, sem_w1,
               sem_w2):
    wid = lax.axis_index("s") * NUM_CORES + lax.axis_index("c")
    idx = (idx_a, idx_b, idx_c)
    rows = (rows_a, rows_b, rows_c)
    sem_g = (sem_g0, sem_g1, sem_g2)
    sem_w = (sem_w0, sem_w1, sem_w2)

    def wait_gather(c):
        pltpu.make_async_copy(wsum_hbm.at[pl.ds(0, CH)], rows[c % 3],
                              sem_g[c % 3]).wait()

    def wait_write(c, g):
        pltpu.make_async_copy(rows[c % 3], out_hbm.at[pl.ds(g * CH, CH)],
                              sem_w[c % 3]).wait()

    pltpu.sync_copy(idx_hbm.at[pl.ds(wid * CH, CH)], idx[0])
    pltpu.async_copy(wsum_hbm.at[idx[0]], rows[0], sem_g[0])
    pltpu.async_copy(idx_hbm.at[pl.ds((NW + wid) * CH, CH)], idx[1], sem_i)

    for c in range(CPW):
        g = c * NW + wid
        b = c % 3
        gn = g + NW

        if c + 1 < CPW:
            @pl.when(gn < NCH)
            def _(c=c, gn=gn):
                nb = (c + 1) % 3
                pltpu.make_async_copy(idx_hbm.at[pl.ds(gn * CH, CH)],
                                      idx[nb], sem_i).wait()
                if c >= 2:
                    wait_write(c - 2, gn - 3 * NW)
                pltpu.async_copy(wsum_hbm.at[idx[nb]], rows[nb], sem_g[nb])
                if c + 2 < CPW:
                    @pl.when(gn + NW < NCH)
                    def _():
                        pltpu.async_copy(
                            idx_hbm.at[pl.ds((gn + NW) * CH, CH)],
                            idx[(c + 2) % 3], sem_i)

        if c < CPW - 1:
            wait_gather(c)
            pltpu.async_copy(rows[b], out_hbm.at[pl.ds(g * CH, CH)],
                             sem_w[b])
        else:
            @pl.when(g < NCH)
            def _(c=c):
                wait_gather(c)

            @pl.when(g < NFULL)
            def _(g=g, b=b):
                pltpu.async_copy(rows[b], out_hbm.at[pl.ds(g * CH, CH)],
                                 sem_w[b])

            @pl.when(g == NFULL)
            def _(g=g, b=b):
                pltpu.sync_copy(rows[b].at[pl.ds(0, TAIL)],
                                out_hbm.at[pl.ds(g * CH, TAIL)])

    wait_write(CPW - 3, (CPW - 3) * NW + wid)
    wait_write(CPW - 2, (CPW - 2) * NW + wid)
    g24 = (CPW - 1) * NW + wid

    @pl.when(g24 < NFULL)
    def _():
        wait_write(CPW - 1, g24)

    @pl.when(g24 >= NCH)
    def _():
        wait_write(CPW - 4, (CPW - 4) * NW + wid)


def kernel(in_degree, out_degree, W_in, W_out):
    pad = PAD_N - N_ROWS
    ind = jnp.pad(in_degree.astype(jnp.int32), (0, pad)).reshape(NCH, CH)
    outd = jnp.pad(out_degree.astype(jnp.int32), (0, pad)).reshape(NCH, CH)
    wsum, idxc = pl.pallas_call(
        _prep_body,
        out_shape=[
            jax.ShapeDtypeStruct((VOCAB, VOCAB, D), jnp.float32),
            jax.ShapeDtypeStruct((NCH, CH), jnp.int32),
        ],
    )(W_in, W_out, ind, outd)
    return _sc_gather(wsum.reshape(VOCAB * VOCAB, D), idxc.reshape(PAD_N))


# speedup vs baseline: 2.5696x; 1.0065x over previous
import functools

import jax
import jax.numpy as jnp
from jax import lax
from jax.experimental import pallas as pl
from jax.experimental.pallas import tpu as pltpu
from jax.experimental.pallas import tpu_sc as plsc

MAX_DEG = 64
VOCAB = MAX_DEG + 1
D = 128
N_ROWS = 100000
NUM_CORES = 2
NUM_SUBCORES = 16
NW = NUM_CORES * NUM_SUBCORES
CH = 128
NFULL = N_ROWS // CH
TAIL = N_ROWS - NFULL * CH
NCH = NFULL + 1
PAD_N = NCH * CH
CPW = (NCH + NW - 1) // NW


def _prep_body(win_ref, wout_ref, ind_ref, outd_ref, wsum_ref, idx_ref):
    win = win_ref[...]
    wout = wout_ref[...]
    wsum_ref[...] = win[:, None, :] + wout[None, :, :]
    a = jnp.clip(ind_ref[...], 0, MAX_DEG)
    b = jnp.clip(outd_ref[...], 0, MAX_DEG)
    idx_ref[...] = a * VOCAB + b


_mesh = plsc.VectorSubcoreMesh(core_axis_name="c", subcore_axis_name="s")


@functools.partial(
    pl.kernel,
    mesh=_mesh,
    out_type=jax.ShapeDtypeStruct((N_ROWS, D), jnp.float32),
    scratch_types=[
        pltpu.VMEM((CH,), jnp.int32),
        pltpu.VMEM((CH,), jnp.int32),
        pltpu.VMEM((CH,), jnp.int32),
        pltpu.VMEM((CH,), jnp.int32),
        pltpu.VMEM((CH, D), jnp.float32),
        pltpu.VMEM((CH, D), jnp.float32),
        pltpu.VMEM((CH, D), jnp.float32),
        pltpu.VMEM((CH, D), jnp.float32),
        pltpu.SemaphoreType.DMA,
        pltpu.SemaphoreType.DMA,
        pltpu.SemaphoreType.DMA,
        pltpu.SemaphoreType.DMA,
        pltpu.SemaphoreType.DMA,
        pltpu.SemaphoreType.DMA,
        pltpu.SemaphoreType.DMA,
        pltpu.SemaphoreType.DMA,
        pltpu.SemaphoreType.DMA,
    ],
)
def _sc_gather(wsum_hbm, idx_hbm, out_hbm, idx_a, idx_b, idx_c, idx_d,
               rows_a, rows_b, rows_c, rows_d, sem_i, sem_g0, sem_g1, sem_g2,
               sem_g3, sem_w0, sem_w1, sem_w2, sem_w3):
    wid = lax.axis_index("s") * NUM_CORES + lax.axis_index("c")
    idx = (idx_a, idx_b, idx_c, idx_d)
    rows = (rows_a, rows_b, rows_c, rows_d)
    sem_g = (sem_g0, sem_g1, sem_g2, sem_g3)
    sem_w = (sem_w0, sem_w1, sem_w2, sem_w3)
    M = 4

    def wait_gather(c):
        pltpu.make_async_copy(wsum_hbm.at[pl.ds(0, CH)], rows[c % M],
                              sem_g[c % M]).wait()

    def wait_write(c, g):
        pltpu.make_async_copy(rows[c % M], out_hbm.at[pl.ds(g * CH, CH)],
                              sem_w[c % M]).wait()

    pltpu.sync_copy(idx_hbm.at[pl.ds(wid * CH, CH)], idx[0])
    pltpu.async_copy(wsum_hbm.at[idx[0]], rows[0], sem_g[0])
    pltpu.async_copy(idx_hbm.at[pl.ds((NW + wid) * CH, CH)], idx[1], sem_i)
    pltpu.make_async_copy(idx_hbm.at[pl.ds((NW + wid) * CH, CH)], idx[1],
                          sem_i).wait()
    pltpu.async_copy(wsum_hbm.at[idx[1]], rows[1], sem_g[1])
    pltpu.async_copy(idx_hbm.at[pl.ds((2 * NW + wid) * CH, CH)], idx[2],
                     sem_i)

    for c in range(CPW):
        g = c * NW + wid
        b = c % M
        g2 = g + 2 * NW

        if c + 2 < CPW:
            @pl.when(g2 < NCH)
            def _(c=c, g2=g2):
                s = (c + 2) % M
                pltpu.make_async_copy(idx_hbm.at[pl.ds(g2 * CH, CH)],
                                      idx[s], sem_i).wait()
                if c >= 2:
                    wait_write(c - 2, g2 - 4 * NW)
                pltpu.async_copy(wsum_hbm.at[idx[s]], rows[s], sem_g[s])
                if c + 3 < CPW:
                    @pl.when(g2 + NW < NCH)
                    def _():
                        pltpu.async_copy(
                            idx_hbm.at[pl.ds((g2 + NW) * CH, CH)],
                            idx[(c + 3) % M], sem_i)

        if c < CPW - 1:
            wait_gather(c)
            pltpu.async_copy(rows[b], out_hbm.at[pl.ds(g * CH, CH)],
                             sem_w[b])
        else:
            @pl.when(g < NCH)
            def _(c=c):
                wait_gather(c)

            @pl.when(g < NFULL)
            def _(g=g, b=b):
                pltpu.async_copy(rows[b], out_hbm.at[pl.ds(g * CH, CH)],
                                 sem_w[b])

            @pl.when(g == NFULL)
            def _(g=g, b=b):
                pltpu.sync_copy(rows[b].at[pl.ds(0, TAIL)],
                                out_hbm.at[pl.ds(g * CH, TAIL)])

    wait_write(CPW - 4, (CPW - 4) * NW + wid)
    wait_write(CPW - 3, (CPW - 3) * NW + wid)
    wait_write(CPW - 2, (CPW - 2) * NW + wid)
    g24 = (CPW - 1) * NW + wid

    @pl.when(g24 < NFULL)
    def _():
        wait_write(CPW - 1, g24)

    @pl.when(g24 >= NCH)
    def _():
        wait_write(CPW - 5, (CPW - 5) * NW + wid)


def kernel(in_degree, out_degree, W_in, W_out):
    pad = PAD_N - N_ROWS
    ind = jnp.pad(in_degree.astype(jnp.int32), (0, pad)).reshape(NCH, CH)
    outd = jnp.pad(out_degree.astype(jnp.int32), (0, pad)).reshape(NCH, CH)
    wsum, idxc = pl.pallas_call(
        _prep_body,
        out_shape=[
            jax.ShapeDtypeStruct((VOCAB, VOCAB, D), jnp.float32),
            jax.ShapeDtypeStruct((NCH, CH), jnp.int32),
        ],
    )(W_in, W_out, ind, outd)
    return _sc_gather(wsum.reshape(VOCAB * VOCAB, D), idxc.reshape(PAD_N))
